# baseline (speedup 1.0000x reference)
import functools

import jax
import jax.numpy as jnp
from jax.experimental import pallas as pl
from jax.experimental.pallas import tpu as pltpu


def _gcn_body(nstrip, x_ref, adj_hbm, w_ref, b_ref, out_ref, adj_s, adjb_s, sems):
    n = adj_s.shape[0]
    rows = n // nstrip
    copies = [
        pltpu.make_async_copy(
            adj_hbm.at[pl.ds(i * rows, rows), :],
            adj_s.at[pl.ds(i * rows, rows), :],
            sems.at[i],
        )
        for i in range(nstrip)
    ]
    for c in copies:
        c.start()
    ht = jax.lax.dot_general(w_ref[:], x_ref[:], (((0,), (1,)), ((), ())),
                             preferred_element_type=jnp.float32)
    ones8 = jnp.full((8, rows), 1.0, dtype=jnp.bfloat16)
    cs8 = jnp.zeros((8, n), dtype=jnp.float32)
    for i in range(nstrip):
        copies[i].wait()
        strip = adj_s[i * rows:(i + 1) * rows, :].astype(jnp.bfloat16)
        adjb_s[i * rows:(i + 1) * rows, :] = strip
        cs8 = cs8 + jax.lax.dot_general(ones8, strip, (((1,), (0,)), ((), ())),
                                        preferred_element_type=jnp.float32)
    colsum = cs8[0:1, :]
    dinv = jax.lax.rsqrt(2.0 * colsum + 1.0)
    hht = ht * dinv
    hi = hht.astype(jnp.bfloat16)
    lo = (hht - hi.astype(jnp.float32)).astype(jnp.bfloat16)
    adjb = adjb_s[:]
    st = (jax.lax.dot_general(hi, adjb, (((1,), (0,)), ((), ())),
                              preferred_element_type=jnp.float32) +
          jax.lax.dot_general(lo, adjb, (((1,), (0,)), ((), ())),
                              preferred_element_type=jnp.float32))
    out_ref[:] = jnp.tanh(dinv * (2.0 * st + hht) + b_ref[:])


def kernel(x, adj, W, b):
    n, in_c = x.shape
    out_c = W.shape[1]
    nstrip = 8
    body = functools.partial(_gcn_body, nstrip)
    return pl.pallas_call(
        body,
        in_specs=[
            pl.BlockSpec((n, in_c), lambda: (0, 0)),
            pl.BlockSpec(memory_space=pltpu.MemorySpace.HBM),
            pl.BlockSpec((in_c, out_c), lambda: (0, 0)),
            pl.BlockSpec((out_c, 1), lambda: (0, 0)),
        ],
        out_specs=pl.BlockSpec((out_c, n), lambda: (0, 0)),
        out_shape=jax.ShapeDtypeStruct((out_c, n), jnp.float32),
        scratch_shapes=[
            pltpu.VMEM((n, n), jnp.float32),
            pltpu.VMEM((n, n), jnp.bfloat16),
            pltpu.SemaphoreType.DMA((nstrip,)),
        ],
    )(x, adj, W, b.reshape(out_c, 1))
